# baseline (speedup 1.0000x reference)
import functools

import jax
import jax.numpy as jnp
from jax import lax
from jax.experimental import pallas as pl
from jax.experimental.pallas import tpu as pltpu
from jax.experimental.pallas import tpu_sc as plsc

NUM_CORES = 2
NUM_SUBCORES = 16
NUM_WORKERS = NUM_CORES * NUM_SUBCORES
LANES = 16
IDX_CHUNK = 128
SUB = 256
TC_LANE_BLK = 4096
N_SLOTS = 4


def _pack_offset(num: int) -> int:
  h = 1
  while h * N_SLOTS < num:
    h *= 2
  return h


@functools.lru_cache(maxsize=None)
def _build_tc_pack(k_dim: int, num: int):
  h = _pack_offset(num)
  assert h % TC_LANE_BLK == 0
  row_w = 2 * k_dim
  wps = k_dim // 2
  stacked = N_SLOTS * k_dim
  grid = h // TC_LANE_BLK
  off_b = h // TC_LANE_BLK
  last_blk = (num - 1) // TC_LANE_BLK

  def body(*refs):
    in_refs, o_ref = refs[:N_SLOTS], refs[N_SLOTS]
    c = jnp.concatenate([r[...] for r in in_refs], axis=0)
    kk = lax.broadcasted_iota(jnp.int32, (stacked, row_w), 0)
    cc = lax.broadcasted_iota(jnp.int32, (stacked, row_w), 1)
    base = (cc // wps) * k_dim + 2 * (cc % wps)
    e_even = (kk == base).astype(jnp.float32)
    e_odd = (kk == base + 1).astype(jnp.float32)
    dn = (((0,), (0,)), ((), ()))
    even = jax.lax.dot_general(c, e_even, dn,
                               precision=jax.lax.Precision.DEFAULT,
                               preferred_element_type=jnp.float32)
    odd = jax.lax.dot_general(c, e_odd, dn,
                              precision=jax.lax.Precision.DEFAULT,
                              preferred_element_type=jnp.float32)
    even_bits = jax.lax.bitcast_convert_type(even, jnp.int32)
    odd_bits = jax.lax.bitcast_convert_type(odd, jnp.int32)
    o_ref[...] = lax.shift_right_logical(even_bits, 16) | odd_bits

  def slot_spec(s):
    if s == 0:
      return pl.BlockSpec((k_dim, TC_LANE_BLK), lambda i: (0, i))
    return pl.BlockSpec(
        (k_dim, TC_LANE_BLK),
        lambda i, s=s: (0, jnp.minimum(i + s * off_b, last_blk)))

  return pl.pallas_call(
      body,
      grid=(grid,),
      in_specs=[slot_spec(s) for s in range(N_SLOTS)],
      out_specs=pl.BlockSpec((TC_LANE_BLK, row_w), lambda i: (i, 0)),
      out_shape=jax.ShapeDtypeStruct((h, row_w), jnp.int32),
  )


@functools.lru_cache(maxsize=None)
def _build_sc_kernel(batch: int, k_dim: int, ent_h: int, rel_h: int):
  assert batch % NUM_WORKERS == 0
  b_per_w = batch // NUM_WORKERS
  assert b_per_w % SUB == 0 and SUB % IDX_CHUNK == 0 and SUB % LANES == 0
  n_sub = b_per_w // SUB
  n_chunks = SUB // IDX_CHUNK
  n_groups = SUB // LANES
  row_w = 2 * k_dim
  wps = k_dim // 2

  mesh = plsc.VectorSubcoreMesh(
      core_axis_name="c", subcore_axis_name="s",
      num_cores=NUM_CORES, num_subcores=NUM_SUBCORES)

  @functools.partial(
      pl.kernel,
      mesh=mesh,
      compiler_params=pltpu.CompilerParams(
          needs_layout_passes=False, use_tc_tiling_on_sc=True),
      out_type=jax.ShapeDtypeStruct((batch,), jnp.float32),
      scratch_types=[
          pltpu.VMEM((b_per_w,), jnp.int32),
          pltpu.VMEM((b_per_w,), jnp.int32),
          pltpu.VMEM((b_per_w,), jnp.int32),
          pltpu.VMEM((SUB,), jnp.int32),
          pltpu.VMEM((SUB,), jnp.int32),
          pltpu.VMEM((SUB,), jnp.int32),
          pltpu.VMEM((SUB, row_w), jnp.int32),
          pltpu.VMEM((SUB, row_w), jnp.int32),
          pltpu.VMEM((SUB, row_w), jnp.int32),
          pltpu.VMEM((b_per_w,), jnp.float32),
          pltpu.SemaphoreType.DMA,
      ],
  )
  def sc_kernel(s_idx_hbm, r_idx_hbm, o_idx_hbm, ent_hbm, rel_hbm, out_hbm,
                s_idx_v, r_idx_v, o_idx_v, s_pid_v, r_pid_v, o_pid_v,
                s_v, r_v, o_v, out_v, sem):
    wid = lax.axis_index("s") * NUM_CORES + lax.axis_index("c")
    base = wid * b_per_w

    pltpu.sync_copy(s_idx_hbm.at[pl.ds(base, b_per_w)], s_idx_v)
    pltpu.sync_copy(r_idx_hbm.at[pl.ds(base, b_per_w)], r_idx_v)
    pltpu.sync_copy(o_idx_hbm.at[pl.ds(base, b_per_w)], o_idx_v)

    lanes = lax.iota(jnp.int32, LANES)
    zeros = jnp.zeros((LANES,), jnp.int32)

    def slot_of(ids, h):
      s = zeros
      for j in range(1, N_SLOTS):
        s = s + jnp.where(ids >= j * h, 1, 0).astype(jnp.int32)
      return s

    for sb in range(n_sub):
      sb_off = sb * SUB
      for v in range(SUB // LANES):
        sl_src = pl.ds(sb_off + v * LANES, LANES)
        sl_dst = pl.ds(v * LANES, LANES)
        s_pid_v[sl_dst] = s_idx_v[sl_src] - slot_of(s_idx_v[sl_src], ent_h) * ent_h
        r_pid_v[sl_dst] = r_idx_v[sl_src] - slot_of(r_idx_v[sl_src], rel_h) * rel_h
        o_pid_v[sl_dst] = o_idx_v[sl_src] - slot_of(o_idx_v[sl_src], ent_h) * ent_h

      copies = []
      for j in range(n_chunks):
        sl = pl.ds(j * IDX_CHUNK, IDX_CHUNK)
        copies.append(pltpu.async_copy(ent_hbm.at[s_pid_v.at[sl]], s_v.at[sl], sem))
        copies.append(pltpu.async_copy(rel_hbm.at[r_pid_v.at[sl]], r_v.at[sl], sem))
        copies.append(pltpu.async_copy(ent_hbm.at[o_pid_v.at[sl]], o_v.at[sl], sem))
      for c in copies:
        c.wait()

      def group_body(g, carry):
        rows = g * LANES + lanes
        sl_ids = pl.ds(sb_off + g * LANES, LANES)
        s_col0 = slot_of(s_idx_v[sl_ids], ent_h) * wps
        r_col0 = slot_of(r_idx_v[sl_ids], rel_h) * wps
        o_col0 = slot_of(o_idx_v[sl_ids], ent_h) * wps
        acc = jnp.zeros((LANES,), jnp.float32)
        for w in range(wps):
          ws = plsc.load_gather(s_v, [rows, s_col0 + w])
          wr = plsc.load_gather(r_v, [rows, r_col0 + w])
          wo = plsc.load_gather(o_v, [rows, o_col0 + w])
          se, so = plsc.unpack(plsc.bitcast(ws, jnp.bfloat16),
                               format=plsc.PackFormat.INTERLEAVED)
          re, ro = plsc.unpack(plsc.bitcast(wr, jnp.bfloat16),
                               format=plsc.PackFormat.INTERLEAVED)
          oe, oo = plsc.unpack(plsc.bitcast(wo, jnp.bfloat16),
                               format=plsc.PackFormat.INTERLEAVED)
          acc = acc + se * re * oe + so * ro * oo
        out_v[pl.ds(sb_off + g * LANES, LANES)] = acc
        return carry

      lax.fori_loop(0, n_groups, group_body, 0)

    pltpu.sync_copy(out_v, out_hbm.at[pl.ds(base, b_per_w)])

  return sc_kernel


def kernel(inputs, ent_emb, rel_emb):
  batch = inputs.shape[0]
  num_ent, k_dim = ent_emb.shape
  num_rel = rel_emb.shape[0]
  ent_t = ent_emb.T
  rel_t = rel_emb.T
  ent2 = _build_tc_pack(k_dim, num_ent)(*([ent_t] * N_SLOTS))
  rel2 = _build_tc_pack(k_dim, num_rel)(*([rel_t] * N_SLOTS))
  s_idx = inputs[:, 0]
  r_idx = inputs[:, 1]
  o_idx = inputs[:, 2]
  fn = _build_sc_kernel(batch, k_dim, ent2.shape[0], rel2.shape[0])
  return fn(s_idx, r_idx, o_idx, ent2, rel2)


# speedup vs baseline: 1.1034x; 1.1034x over previous
import functools

import jax
import jax.numpy as jnp
from jax import lax
from jax.experimental import pallas as pl
from jax.experimental.pallas import tpu as pltpu
from jax.experimental.pallas import tpu_sc as plsc

NUM_CORES = 2
NUM_SUBCORES = 16
NUM_WORKERS = NUM_CORES * NUM_SUBCORES
LANES = 16
IDX_CHUNK = 128
SUB = 256
TC_LANE_BLK = 16384
N_SLOTS = 4


def _pack_offset(num: int) -> int:
  h = 1
  while h * N_SLOTS < num:
    h *= 2
  return h


@functools.lru_cache(maxsize=None)
def _build_tc_pack(k_dim: int, num: int):
  h = _pack_offset(num)
  assert h % TC_LANE_BLK == 0
  row_w = 2 * k_dim
  wps = k_dim // 2
  stacked = N_SLOTS * k_dim
  grid = h // TC_LANE_BLK
  off_b = h // TC_LANE_BLK
  last_blk = (num - 1) // TC_LANE_BLK

  def body(*refs):
    in_refs, o_ref = refs[:N_SLOTS], refs[N_SLOTS]
    c = jnp.concatenate([r[...] for r in in_refs], axis=0)
    kk = lax.broadcasted_iota(jnp.int32, (stacked, row_w), 0)
    cc = lax.broadcasted_iota(jnp.int32, (stacked, row_w), 1)
    base = (cc // wps) * k_dim + 2 * (cc % wps)
    e_even = (kk == base).astype(jnp.float32)
    e_odd = (kk == base + 1).astype(jnp.float32)
    dn = (((0,), (0,)), ((), ()))
    even = jax.lax.dot_general(c, e_even, dn,
                               precision=jax.lax.Precision.DEFAULT,
                               preferred_element_type=jnp.float32)
    odd = jax.lax.dot_general(c, e_odd, dn,
                              precision=jax.lax.Precision.DEFAULT,
                              preferred_element_type=jnp.float32)
    even_bits = jax.lax.bitcast_convert_type(even, jnp.int32)
    odd_bits = jax.lax.bitcast_convert_type(odd, jnp.int32)
    o_ref[...] = lax.shift_right_logical(even_bits, 16) | odd_bits

  def slot_spec(s):
    if s == 0:
      return pl.BlockSpec((k_dim, TC_LANE_BLK), lambda i: (0, i))
    return pl.BlockSpec(
        (k_dim, TC_LANE_BLK),
        lambda i, s=s: (0, jnp.minimum(i + s * off_b, last_blk)))

  return pl.pallas_call(
      body,
      grid=(grid,),
      in_specs=[slot_spec(s) for s in range(N_SLOTS)],
      out_specs=pl.BlockSpec((TC_LANE_BLK, row_w), lambda i: (i, 0)),
      out_shape=jax.ShapeDtypeStruct((h, row_w), jnp.int32),
  )


@functools.lru_cache(maxsize=None)
def _build_sc_kernel(batch: int, k_dim: int, ent_h: int, rel_h: int):
  assert batch % NUM_WORKERS == 0
  b_per_w = batch // NUM_WORKERS
  assert b_per_w % SUB == 0 and SUB % IDX_CHUNK == 0 and SUB % LANES == 0
  n_sub = b_per_w // SUB
  n_chunks = SUB // IDX_CHUNK
  n_groups = SUB // LANES
  row_w = 2 * k_dim
  wps = k_dim // 2

  mesh = plsc.VectorSubcoreMesh(
      core_axis_name="c", subcore_axis_name="s",
      num_cores=NUM_CORES, num_subcores=NUM_SUBCORES)

  @functools.partial(
      pl.kernel,
      mesh=mesh,
      compiler_params=pltpu.CompilerParams(
          needs_layout_passes=False, use_tc_tiling_on_sc=True),
      out_type=jax.ShapeDtypeStruct((batch,), jnp.float32),
      scratch_types=[
          pltpu.VMEM((b_per_w,), jnp.int32),
          pltpu.VMEM((b_per_w,), jnp.int32),
          pltpu.VMEM((b_per_w,), jnp.int32),
          pltpu.VMEM((SUB,), jnp.int32),
          pltpu.VMEM((SUB,), jnp.int32),
          pltpu.VMEM((SUB,), jnp.int32),
          pltpu.VMEM((SUB, row_w), jnp.int32),
          pltpu.VMEM((SUB, row_w), jnp.int32),
          pltpu.VMEM((SUB, row_w), jnp.int32),
          pltpu.VMEM((b_per_w,), jnp.float32),
          pltpu.SemaphoreType.DMA,
      ],
  )
  def sc_kernel(s_idx_hbm, r_idx_hbm, o_idx_hbm, ent_hbm, rel_hbm, out_hbm,
                s_idx_v, r_idx_v, o_idx_v, s_pid_v, r_pid_v, o_pid_v,
                s_v, r_v, o_v, out_v, sem):
    wid = lax.axis_index("s") * NUM_CORES + lax.axis_index("c")
    base = wid * b_per_w

    pltpu.sync_copy(s_idx_hbm.at[pl.ds(base, b_per_w)], s_idx_v)
    pltpu.sync_copy(r_idx_hbm.at[pl.ds(base, b_per_w)], r_idx_v)
    pltpu.sync_copy(o_idx_hbm.at[pl.ds(base, b_per_w)], o_idx_v)

    lanes = lax.iota(jnp.int32, LANES)
    zeros = jnp.zeros((LANES,), jnp.int32)

    def slot_of(ids, h):
      s = zeros
      for j in range(1, N_SLOTS):
        s = s + jnp.where(ids >= j * h, 1, 0).astype(jnp.int32)
      return s

    for sb in range(n_sub):
      sb_off = sb * SUB
      for v in range(SUB // LANES):
        sl_src = pl.ds(sb_off + v * LANES, LANES)
        sl_dst = pl.ds(v * LANES, LANES)
        s_pid_v[sl_dst] = s_idx_v[sl_src] - slot_of(s_idx_v[sl_src], ent_h) * ent_h
        r_pid_v[sl_dst] = r_idx_v[sl_src] - slot_of(r_idx_v[sl_src], rel_h) * rel_h
        o_pid_v[sl_dst] = o_idx_v[sl_src] - slot_of(o_idx_v[sl_src], ent_h) * ent_h

      copies = []
      for j in range(n_chunks):
        sl = pl.ds(j * IDX_CHUNK, IDX_CHUNK)
        copies.append(pltpu.async_copy(ent_hbm.at[s_pid_v.at[sl]], s_v.at[sl], sem))
        copies.append(pltpu.async_copy(rel_hbm.at[r_pid_v.at[sl]], r_v.at[sl], sem))
        copies.append(pltpu.async_copy(ent_hbm.at[o_pid_v.at[sl]], o_v.at[sl], sem))
      for c in copies:
        c.wait()

      def group_body(g, carry):
        rows = g * LANES + lanes
        sl_ids = pl.ds(sb_off + g * LANES, LANES)
        s_col0 = slot_of(s_idx_v[sl_ids], ent_h) * wps
        r_col0 = slot_of(r_idx_v[sl_ids], rel_h) * wps
        o_col0 = slot_of(o_idx_v[sl_ids], ent_h) * wps
        acc = jnp.zeros((LANES,), jnp.float32)
        for w in range(wps):
          ws = plsc.load_gather(s_v, [rows, s_col0 + w])
          wr = plsc.load_gather(r_v, [rows, r_col0 + w])
          wo = plsc.load_gather(o_v, [rows, o_col0 + w])
          se, so = plsc.unpack(plsc.bitcast(ws, jnp.bfloat16),
                               format=plsc.PackFormat.INTERLEAVED)
          re, ro = plsc.unpack(plsc.bitcast(wr, jnp.bfloat16),
                               format=plsc.PackFormat.INTERLEAVED)
          oe, oo = plsc.unpack(plsc.bitcast(wo, jnp.bfloat16),
                               format=plsc.PackFormat.INTERLEAVED)
          acc = acc + se * re * oe + so * ro * oo
        out_v[pl.ds(sb_off + g * LANES, LANES)] = acc
        return carry

      lax.fori_loop(0, n_groups, group_body, 0)

    pltpu.sync_copy(out_v, out_hbm.at[pl.ds(base, b_per_w)])

  return sc_kernel


def kernel(inputs, ent_emb, rel_emb):
  batch = inputs.shape[0]
  num_ent, k_dim = ent_emb.shape
  num_rel = rel_emb.shape[0]
  ent_t = ent_emb.T
  rel_t = rel_emb.T
  ent2 = _build_tc_pack(k_dim, num_ent)(*([ent_t] * N_SLOTS))
  rel2 = _build_tc_pack(k_dim, num_rel)(*([rel_t] * N_SLOTS))
  s_idx = inputs[:, 0]
  r_idx = inputs[:, 1]
  o_idx = inputs[:, 2]
  fn = _build_sc_kernel(batch, k_dim, ent2.shape[0], rel2.shape[0])
  return fn(s_idx, r_idx, o_idx, ent2, rel2)
